# Initial kernel scaffold; baseline (speedup 1.0000x reference)
import jax
import jax.numpy as jnp
from jax.experimental import pallas as pl


def kernel(maps, edge_index, num_nodes):
    raise NotImplementedError("write your pallas kernel here")


# speedup vs baseline: 2.7986x; 2.7986x over previous
import jax
import jax.numpy as jnp
from jax import lax
from jax.experimental import pallas as pl

_D = 4
_DD = 16
_NS_ITERS = 14


def _mm(P, Q):
    rows = []
    for i in range(4):
        for k in range(4):
            s = None
            for j in range(4):
                t = P[4 * i + j:4 * i + j + 1] * Q[4 * j + k:4 * j + k + 1]
                s = t if s is None else s + t
            rows.append(s)
    return jnp.concatenate(rows, axis=0)


def _mmT(P, Q):
    rows = []
    for i in range(4):
        for k in range(4):
            s = None
            for j in range(4):
                t = P[4 * j + i:4 * j + i + 1] * Q[4 * j + k:4 * j + k + 1]
                s = t if s is None else s + t
            rows.append(s)
    return jnp.concatenate(rows, axis=0)


def _edge_contrib_kernel(maps_ref, out_ref):
    A = maps_ref[...]
    out_ref[...] = _mmT(A, A)


def _scatter_kernel(ctr_ref, rows_ref, acc_ref):
    @pl.when(pl.program_id(0) == 0)
    def _():
        acc_ref[...] = jnp.zeros_like(acc_ref)

    bs = ctr_ref.shape[0]

    def body(b, carry):
        idx = rows_ref[0, b, 0]
        acc_ref[pl.ds(idx, 1), :] = (acc_ref[pl.ds(idx, 1), :]
                                     + ctr_ref[pl.ds(b, 1), :])
        return carry

    lax.fori_loop(0, bs, body, 0)


def _ns_kernel(md_ref, s_ref, dval_ref, drow_ref, dcol_ref):
    Ad = md_ref[...]
    bn = Ad.shape[1]
    l = lax.broadcasted_iota(jnp.int32, (16, 1), 0)
    eye = (l % 5 == 0).astype(Ad.dtype)
    A = Ad + eye
    c = A[0:1] + A[5:6] + A[10:11] + A[15:16]
    cinv = 1.0 / c
    Y = A * cinv
    Z = eye + jnp.zeros_like(A)
    for _ in range(_NS_ITERS):
        W = _mm(Z, Y)
        T3 = 3.0 * eye - W
        Y = 0.5 * _mm(Y, T3)
        Z = 0.5 * _mm(T3, Z)
    S = Z * lax.rsqrt(c)
    s_ref[...] = S
    Dc = jnp.clip(_mm(_mm(S, Ad), S), -1.0, 1.0)
    dval_ref[...] = Dc
    node = bn * pl.program_id(0) + lax.broadcasted_iota(jnp.int32, (1, bn), 1)
    ii = l // 4
    kk = l % 4
    drow_ref[...] = 4 * node + ii
    dcol_ref[...] = 4 * node + kk


def _tri_kernel(mapl_ref, mapr_ref, sl_ref, sr_ref, rows_ref, cols_ref,
                vij_ref, vji_ref, rij_ref, cij_ref, rji_ref, cji_ref):
    L = mapl_ref[...]
    R = mapr_ref[...]
    SL = sl_ref[...]
    SR = sr_ref[...]
    T0 = _mmT(L, R)
    Vc = jnp.clip(_mm(_mm(SL, T0), SR), -1.0, 1.0)
    vij_ref[...] = -Vc
    vji_rows = []
    for i in range(4):
        for k in range(4):
            vji_rows.append(-Vc[4 * k + i:4 * k + i + 1])
    vji_ref[...] = jnp.concatenate(vji_rows, axis=0)
    l = lax.broadcasted_iota(jnp.int32, (16, 1), 0)
    ii = l // 4
    kk = l % 4
    r2 = rows_ref[0]
    c2 = cols_ref[0]
    rij_ref[...] = 4 * r2 + ii
    cij_ref[...] = 4 * c2 + kk
    rji_ref[...] = 4 * c2 + ii
    cji_ref[...] = 4 * r2 + kk


def _flat(xT):
    return xT.T.reshape(-1)


def _build(maps, edge_index, num_nodes, n, B1, Bs, Bn, B3, Np):
    twoE = maps.shape[0]
    E = twoE // 2
    f32 = maps.dtype

    mapsT = maps.reshape(twoE, _DD).T

    contribT = pl.pallas_call(
        _edge_contrib_kernel,
        grid=(twoE // B1,),
        in_specs=[pl.BlockSpec((_DD, B1), lambda i: (0, i))],
        out_specs=pl.BlockSpec((_DD, B1), lambda i: (0, i)),
        out_shape=jax.ShapeDtypeStruct((_DD, twoE), f32),
    )(mapsT)
    contrib = contribT.T

    rows_all = edge_index[0]
    rows3 = rows_all.reshape(twoE // Bs, Bs, 1)
    macc = pl.pallas_call(
        _scatter_kernel,
        grid=(twoE // Bs,),
        in_specs=[
            pl.BlockSpec((Bs, _DD), lambda i: (i, 0)),
            pl.BlockSpec((1, Bs, 1), lambda i: (i, 0, 0)),
        ],
        out_specs=pl.BlockSpec((n, _DD), lambda i: (0, 0)),
        out_shape=jax.ShapeDtypeStruct((n, _DD), f32),
    )(contrib, rows3)

    mdT = jnp.concatenate(
        [macc, jnp.zeros((Np - n, _DD), f32)], axis=0).T

    ST, dvT, drT, dcT = pl.pallas_call(
        _ns_kernel,
        grid=(Np // Bn,),
        in_specs=[pl.BlockSpec((_DD, Bn), lambda i: (0, i))],
        out_specs=[pl.BlockSpec((_DD, Bn), lambda i: (0, i))] * 4,
        out_shape=[
            jax.ShapeDtypeStruct((_DD, Np), f32),
            jax.ShapeDtypeStruct((_DD, Np), f32),
            jax.ShapeDtypeStruct((_DD, Np), jnp.int32),
            jax.ShapeDtypeStruct((_DD, Np), jnp.int32),
        ],
    )(mdT)

    Sn = ST[:, :n]
    rowE = edge_index[0, :E]
    colE = edge_index[1, :E]
    SL = jnp.take(Sn, rowE, axis=1)
    SR = jnp.take(Sn, colE, axis=1)
    r3 = rowE.reshape(E // B3, 1, B3)
    c3 = colE.reshape(E // B3, 1, B3)

    off = E // B3
    vijT, vjiT, rijT, cijT, rjiT, cjiT = pl.pallas_call(
        _tri_kernel,
        grid=(E // B3,),
        in_specs=[
            pl.BlockSpec((_DD, B3), lambda i: (0, i)),
            pl.BlockSpec((_DD, B3), lambda i: (0, i + off)),
            pl.BlockSpec((_DD, B3), lambda i: (0, i)),
            pl.BlockSpec((_DD, B3), lambda i: (0, i)),
            pl.BlockSpec((1, 1, B3), lambda i: (i, 0, 0)),
            pl.BlockSpec((1, 1, B3), lambda i: (i, 0, 0)),
        ],
        out_specs=[pl.BlockSpec((_DD, B3), lambda i: (0, i))] * 6,
        out_shape=[
            jax.ShapeDtypeStruct((_DD, E), f32),
            jax.ShapeDtypeStruct((_DD, E), f32),
            jax.ShapeDtypeStruct((_DD, E), jnp.int32),
            jax.ShapeDtypeStruct((_DD, E), jnp.int32),
            jax.ShapeDtypeStruct((_DD, E), jnp.int32),
            jax.ShapeDtypeStruct((_DD, E), jnp.int32),
        ],
    )(mapsT, mapsT, SL, SR, r3, c3)

    n_residual = num_nodes - n
    rows = jnp.concatenate([_flat(drT[:, :n]), _flat(rijT), _flat(rjiT)]) \
        + n_residual
    cols = jnp.concatenate([_flat(dcT[:, :n]), _flat(cijT), _flat(cjiT)])
    vals = jnp.concatenate([_flat(dvT[:, :n]), _flat(vijT), _flat(vjiT)])
    indices = jnp.stack([rows, cols], axis=0)
    return indices, vals


def kernel(maps, edge_index, num_nodes):
    return _build(maps, edge_index, num_nodes, n=50000,
                  B1=3200, Bs=2000, Bn=3200, B3=3200, Np=51200)
